# baseline (speedup 1.0000x reference)
import functools
import jax
import jax.numpy as jnp
from jax import lax
from jax.experimental import pallas as pl
from jax.experimental.pallas import tpu as pltpu, tpu_sc as plsc

_R = 1024
_C = 26
_SIZE = 1000
_TR = _SIZE // 8
_NSLAB = _C * _TR
_SLAB = 8192
_CHUNK = 2 * _SLAB
_NBUF = 2


def _divmod125(s):
    c = (s * 8389) >> 20
    return c, s - c * _TR


def _sc_body(idx_hbm, out_hbm, idx_v, addr_v, trkey_v, *bufs_and_sems):
    bufs = bufs_and_sems[:_NBUF]
    sems = bufs_and_sems[_NBUF:]
    nc = 2
    w = lax.axis_index("s") * nc + lax.axis_index("c")

    base = jnp.where(w < 25, 102 * w, 100 * w + 50)
    cb = base // 2
    c0, _tr0 = _divmod125(base)

    pltpu.sync_copy(idx_hbm.at[pl.ds(c0 * _R, 2 * _R)], idx_v)

    lane = lax.iota(jnp.int32, 16)
    zeros16 = jnp.zeros((16,), jnp.float32)
    ones16 = jnp.ones((16,), jnp.float32)

    def _cache_body(v, _):
        iv = idx_v[pl.ds(v * 16, 16)]
        tc_of = (v & 63) >> 3
        g_of = v & 7
        addr_v[pl.ds(v * 16, 16)] = (
            tc_of * 1024 + (iv & 7) * 128 + g_of * 16 + lane
        )
        trkey_v[pl.ds(v * 16, 16)] = iv >> 3
        return _

    lax.fori_loop(0, 128, _cache_body, 0)

    def _zero_body(i, _):
        for b in range(_NBUF):
            bufs[b][pl.ds(i * 16, 16)] = zeros16
        return _

    lax.fori_loop(0, _CHUNK // 16, _zero_body, 0)

    def _chunk_scatter(buf, q, val):
        for half in range(2):
            c, tr = _divmod125(2 * q + half)
            cl = c - c0
            trv = jnp.full((16,), tr, jnp.int32)
            off = half * _SLAB

            def _tc_body(tc, _):
                for g in range(8):
                    vb = (cl * 64 + tc * 8 + g) * 16
                    kv = trkey_v[pl.ds(vb, 16)]
                    ad = addr_v[pl.ds(vb, 16)]
                    plsc.store_scatter(buf, [ad + off], val, mask=kv == trv)
                return _

            lax.fori_loop(0, 8, _tc_body, 0)

    def _start(buf, sem, q):
        return pltpu.async_copy(
            buf, out_hbm.at[pl.ds(q * _CHUNK, _CHUNK)], sem
        )

    def _drain(buf, sem):
        pltpu.make_async_copy(buf, out_hbm.at[pl.ds(0, _CHUNK)], sem).wait()

    for b in range(_NBUF):
        _chunk_scatter(bufs[b], cb + b, ones16)
        _start(bufs[b], sems[b], cb + b)

    def _loop_body(j, _):
        for b in range(_NBUF):
            i = _NBUF * j + b
            _drain(bufs[b], sems[b])
            _chunk_scatter(bufs[b], cb + i - _NBUF, zeros16)
            _chunk_scatter(bufs[b], cb + i, ones16)
            _start(bufs[b], sems[b], cb + i)
        return _

    lax.fori_loop(1, 48 // _NBUF, _loop_body, 0)

    for b in range(2):
        i = 48 + b
        _drain(bufs[b], sems[b])
        _chunk_scatter(bufs[b], cb + i - _NBUF, zeros16)
        _chunk_scatter(bufs[b], cb + i, ones16)
        _start(bufs[b], sems[b], cb + i)

    @pl.when(w < 25)
    def _extra():
        _drain(bufs[0], sems[0])
        _chunk_scatter(bufs[0], cb + 48, zeros16)
        _chunk_scatter(bufs[0], cb + 50, ones16)
        _start(bufs[0], sems[0], cb + 50)

    for b in range(_NBUF):
        _drain(bufs[b], sems[b])


_sc_onehot = functools.partial(
    pl.kernel,
    mesh=plsc.VectorSubcoreMesh(core_axis_name="c", subcore_axis_name="s"),
    out_type=jax.ShapeDtypeStruct((_R * _C * _SIZE,), jnp.float32),
    compiler_params=pltpu.CompilerParams(needs_layout_passes=False),
    scratch_types=[
        pltpu.VMEM((2 * _R,), jnp.int32),
        pltpu.VMEM((2 * _R,), jnp.int32),
        pltpu.VMEM((2 * _R,), jnp.int32),
        *([pltpu.VMEM((_CHUNK,), jnp.float32)] * _NBUF),
        *([pltpu.SemaphoreType.DMA] * _NBUF),
    ],
)(_sc_body)


def kernel(x, size):
    del size
    idx_t = x.astype(jnp.int32).T.reshape(_C * _R)
    idx_t = jnp.pad(idx_t, (0, _R))
    out = _sc_onehot(idx_t)
    return (
        out.reshape(_C, _TR, 8, 8, 128)
        .transpose(2, 4, 0, 1, 3)
        .reshape(_R, _C, _SIZE)
    )


# speedup vs baseline: 2.0069x; 2.0069x over previous
import functools
import jax
import jax.numpy as jnp
from jax import lax
from jax.experimental import pallas as pl
from jax.experimental.pallas import tpu as pltpu, tpu_sc as plsc

_R = 1024
_C = 26
_SIZE = 1000
_TR = _SIZE // 8
_NSLAB = _C * _TR
_SLAB = 8192
_NBUF = 4


def _divmod125(s):
    c = (s * 8389) >> 20
    return c, s - c * _TR


def _slab_compute(idx_v, buf, c_local, tr):
    k0 = tr * 8

    def _tc_body(tc, _):
        base = c_local * _R + tc * 128
        ivs = [idx_v[pl.ds(base + g * 16, 16)] for g in range(8)]
        for ks in range(8):
            kvec = jnp.full((16,), k0 + ks, jnp.int32)
            for g in range(8):
                buf[pl.ds(tc * 1024 + ks * 128 + g * 16, 16)] = (
                    ivs[g] == kvec
                ).astype(jnp.float32)
        return _

    lax.fori_loop(0, 8, _tc_body, 0)


def _advance(c, tr):
    wrap = (tr + 1 >= _TR).astype(jnp.int32)
    return c + wrap, (tr + 1) - wrap * _TR


def _sc_body(idx_hbm, out_hbm, idx_v, *bufs_and_sems):
    bufs = bufs_and_sems[:_NBUF]
    sems = bufs_and_sems[_NBUF:]
    nc = 2
    w = lax.axis_index("s") * nc + lax.axis_index("c")

    base = 101 * w + jnp.minimum(w, 18)
    c0, tr0 = _divmod125(base)

    pltpu.sync_copy(idx_hbm.at[pl.ds(c0 * _R, 2 * _R)], idx_v)

    def _start(buf, sem, s):
        return pltpu.async_copy(buf, out_hbm.at[pl.ds(s * _SLAB, _SLAB)], sem)

    def _drain(buf, sem):
        pltpu.make_async_copy(buf, out_hbm.at[pl.ds(0, _SLAB)], sem).wait()

    c, tr = c0, tr0
    for b in range(_NBUF):
        _slab_compute(idx_v, bufs[b], c - c0, tr)
        _start(bufs[b], sems[b], base + b)
        c, tr = _advance(c, tr)

    def _loop_body(j, carry):
        c, tr = carry
        for b in range(_NBUF):
            s = base + _NBUF * j + b
            _drain(bufs[b], sems[b])
            _slab_compute(idx_v, bufs[b], c - c0, tr)
            _start(bufs[b], sems[b], s)
            c, tr = _advance(c, tr)
        return c, tr

    c, tr = lax.fori_loop(1, 25, _loop_body, (c, tr))

    _drain(bufs[0], sems[0])
    _slab_compute(idx_v, bufs[0], c - c0, tr)
    _start(bufs[0], sems[0], base + 100)
    c, tr = _advance(c, tr)

    @pl.when(w < 18)
    def _extra():
        _drain(bufs[1], sems[1])
        _slab_compute(idx_v, bufs[1], c - c0, tr)
        _start(bufs[1], sems[1], base + 101)

    for b in range(_NBUF):
        _drain(bufs[b], sems[b])


_sc_onehot = functools.partial(
    pl.kernel,
    mesh=plsc.VectorSubcoreMesh(core_axis_name="c", subcore_axis_name="s"),
    out_type=jax.ShapeDtypeStruct((_R * _C * _SIZE,), jnp.float32),
    compiler_params=pltpu.CompilerParams(needs_layout_passes=False),
    scratch_types=[
        pltpu.VMEM((2 * _R,), jnp.int32),
        *([pltpu.VMEM((_SLAB,), jnp.float32)] * _NBUF),
        *([pltpu.SemaphoreType.DMA] * _NBUF),
    ],
)(_sc_body)


def kernel(x, size):
    del size
    idx_t = x.astype(jnp.int32).T.reshape(_C * _R)
    idx_t = jnp.pad(idx_t, (0, _R))
    out = _sc_onehot(idx_t)
    return (
        out.reshape(_C, _TR, 8, 8, 128)
        .transpose(2, 4, 0, 1, 3)
        .reshape(_R, _C, _SIZE)
    )
